# baseline (speedup 1.0000x reference)
import functools

import jax
import jax.numpy as jnp
from jax import lax
from jax.experimental import pallas as pl
from jax.experimental.pallas import tpu as pltpu
from jax.experimental.pallas import tpu_sc as plsc

NC = 2
NS = 16
NW = NC * NS
L = 16


def _proj_body(x_ref, w_ref, b_ref, o_ref):
    o_ref[...] = (
        jnp.dot(x_ref[...], w_ref[...], preferred_element_type=jnp.float32)
        + b_ref[...]
    )


def _node_proj(x_p, w_bd, b_t):
    np8, dxp = x_p.shape
    hp = w_bd.shape[1]
    return pl.pallas_call(
        _proj_body,
        grid=(1,),
        in_specs=[
            pl.BlockSpec((np8, dxp), lambda i: (0, 0)),
            pl.BlockSpec((dxp, hp), lambda i: (0, 0)),
            pl.BlockSpec((1, hp), lambda i: (0, 0)),
        ],
        out_specs=pl.BlockSpec((np8, hp), lambda i: (0, 0)),
        out_shape=jax.ShapeDtypeStruct((np8, hp), jnp.float32),
    )(x_p, w_bd, b_t)


def _edge_proj_body(w1_ref, eat_ref, o_ref):
    o_ref[...] = lax.dot_general(
        w1_ref[...], eat_ref[...],
        (((0,), (0,)), ((), ())),
        preferred_element_type=jnp.float32,
    )


def _edge_proj(ea_t, w1):
    de, e = ea_t.shape
    blk = 32000
    return pl.pallas_call(
        _edge_proj_body,
        grid=(e // blk,),
        in_specs=[
            pl.BlockSpec((de, de), lambda i: (0, 0)),
            pl.BlockSpec((de, blk), lambda i: (0, i)),
        ],
        out_specs=pl.BlockSpec((de, blk), lambda i: (0, i)),
        out_shape=jax.ShapeDtypeStruct((de, e), jnp.float32),
    )(w1, ea_t)


def _banded(ebase):
    k = ebase // 1000
    return (k // 8) * 1000, (k % 8) * L


def _gather_body(ew, p_hbm, src_hbm, qt_hbm, g_hbm,
                 idx0, idx1, rows0, rows1, qt0, qt1, z0, z1,
                 sem_i0, sem_i1, sem_g, sem_w0, sem_w1):
    wid = lax.axis_index("s") * NC + lax.axis_index("c")
    nch = ew // 1000
    idx_b = (idx0, idx1)
    rows_b = (rows0, rows1)
    qt_b = (qt0, qt1)
    z_b = (z0, z1)
    sem_i = (sem_i0, sem_i1)
    sem_w = (sem_w0, sem_w1)
    col = lax.iota(jnp.int32, L)

    def fire_in(ci, b):
        estart = wid * ew + ci * 1000
        pltpu.async_copy(src_hbm.at[pl.ds(estart, 1000)], idx_b[b], sem_i[b])
        pltpu.async_copy(qt_hbm.at[:, pl.ds(estart, 1000)], qt_b[b], sem_i[b])

    fire_in(0, 0)

    @pl.loop(0, nch // 2)
    def _outer(o):
        for b in range(2):
            ci = 2 * o + b

            @pl.when(ci + 1 < nch)
            def _():
                fire_in(ci + 1, 1 - b)

            pltpu.make_async_copy(
                src_hbm.at[pl.ds(0, 1000)], idx_b[b], sem_i[b]
            ).wait()
            pltpu.make_async_copy(
                qt_hbm.at[:, pl.ds(0, 1000)], qt_b[b], sem_i[b]
            ).wait()

            @pl.when(o > 0)
            def _():
                pltpu.make_async_copy(
                    z_b[b], g_hbm.at[pl.ds(0, 1000), pl.ds(0, L)], sem_w[b]
                ).wait()

            copies = [
                pltpu.async_copy(
                    p_hbm.at[idx_b[b].at[pl.ds(j * 128, min(128, 1000 - j * 128))]],
                    rows_b[b].at[pl.ds(j * 128, min(128, 1000 - j * 128))],
                    sem_g,
                )
                for j in range(8)
            ]
            for cp in copies:
                cp.wait()

            @pl.loop(0, 1000, unroll=8)
            def _zrow(t):
                tv = jnp.broadcast_to(t, (L,))
                qcol = plsc.load_gather(qt_b[b], [col, tv])
                prow = plsc.load_gather(rows_b[b], [tv, col])
                z = jnp.maximum(prow + qcol, 0.0)
                plsc.store_scatter(z_b[b], [tv, col], z)

            row0, col0 = _banded(wid * ew + ci * 1000)
            pltpu.async_copy(
                z_b[b],
                g_hbm.at[pl.ds(row0, 1000), pl.ds(col0, L)],
                sem_w[b],
            )

    for b in range(2):
        pltpu.make_async_copy(
            z_b[b], g_hbm.at[pl.ds(0, 1000), pl.ds(0, L)], sem_w[b]
        ).wait()


def _sc_gather(p, src, qt, e):
    ew = e // NW
    mesh = plsc.VectorSubcoreMesh(
        core_axis_name="c", subcore_axis_name="s", num_cores=NC, num_subcores=NS
    )
    return pl.kernel(
        functools.partial(_gather_body, ew),
        out_type=jax.ShapeDtypeStruct((e // 8, 8 * L), jnp.float32),
        mesh=mesh,
        compiler_params=pltpu.CompilerParams(
            use_tc_tiling_on_sc=False, needs_layout_passes=False
        ),
        scratch_types=[
            pltpu.VMEM((1000,), jnp.int32),
            pltpu.VMEM((1000,), jnp.int32),
            pltpu.VMEM((1000, L), jnp.float32),
            pltpu.VMEM((1000, L), jnp.float32),
            pltpu.VMEM((L, 1000), jnp.float32),
            pltpu.VMEM((L, 1000), jnp.float32),
            pltpu.VMEM((1000, L), jnp.float32),
            pltpu.VMEM((1000, L), jnp.float32),
            pltpu.SemaphoreType.DMA,
            pltpu.SemaphoreType.DMA,
            pltpu.SemaphoreType.DMA,
            pltpu.SemaphoreType.DMA,
            pltpu.SemaphoreType.DMA,
        ],
    )(p, src, qt)


def _msg_body(z_ref, w2_ref, b2_ref, o_ref):
    o_ref[...] = (
        jnp.dot(z_ref[...], w2_ref[...], preferred_element_type=jnp.float32)
        + b2_ref[...]
    )


def _msg_mlp(z_pb, w2_bd, b2_t):
    ep8, dep = z_pb.shape
    blk = 4000
    return pl.pallas_call(
        _msg_body,
        grid=(ep8 // blk,),
        in_specs=[
            pl.BlockSpec((blk, dep), lambda i: (i, 0)),
            pl.BlockSpec((dep, dep), lambda i: (0, 0)),
            pl.BlockSpec((1, dep), lambda i: (0, 0)),
        ],
        out_specs=pl.BlockSpec((blk, dep), lambda i: (i, 0)),
        out_shape=jax.ShapeDtypeStruct((ep8, dep), jnp.float32),
    )(z_pb, w2_bd, b2_t)


_BCAST_DNUMS = lax.GatherDimensionNumbers(
    offset_dims=(), collapsed_slice_dims=(0,), start_index_map=(0,)
)


def _bcast_lane(v, b):
    return lax.gather(
        v,
        jnp.full((L, 1), b, jnp.int32),
        _BCAST_DNUMS,
        slice_sizes=(1,),
        mode=lax.GatherScatterMode.PROMISE_IN_BOUNDS,
    )


def _scatter_body(eg, nh, m_hbm, dst_hbm, out_hbm, acc,
                  d0, d1, m0, m1, sem0, sem1):
    wid = lax.axis_index("s") * NC + lax.axis_index("c")
    g = wid // 2
    h = wid % 2
    col = lax.iota(jnp.int32, L)
    neg_inf = jnp.full((L,), -jnp.inf, dtype=jnp.float32)
    d_b = (d0, d1)
    m_b = (m0, m1)
    sem = (sem0, sem1)

    @pl.loop(0, nh + 1)
    def _init(i):
        acc[pl.ds(i * L, L)] = neg_inf

    nch = eg // 1000
    nvec = 1000 // L + 1

    def fire(ci, b):
        ebase = g * eg + ci * 1000
        row0, col0 = _banded(ebase)
        pltpu.async_copy(dst_hbm.at[pl.ds(ebase, 1000)], d_b[b], sem[b])
        pltpu.async_copy(
            m_hbm.at[pl.ds(row0, 1000), pl.ds(col0, L)], m_b[b], sem[b]
        )

    fire(0, 0)

    @pl.loop(0, nch // 2)
    def _outer(o):
        for b in range(2):
            ci = 2 * o + b

            @pl.when(ci + 1 < nch)
            def _():
                fire(ci + 1, 1 - b)

            pltpu.make_async_copy(
                dst_hbm.at[pl.ds(0, 1000)], d_b[b], sem[b]
            ).wait()
            pltpu.make_async_copy(
                m_hbm.at[pl.ds(0, 1000), pl.ds(0, L)], m_b[b], sem[b]
            ).wait()

            @pl.loop(0, nvec)
            def _vec(k):
                eoff = jnp.minimum(k * L, 1000 - L)
                dstv = d_b[b][pl.ds(eoff, L)]
                off = dstv - h * nh
                owned = (off >= 0) & (off < nh)
                base16 = jnp.where(owned, off, nh) * L
                for lane in range(L):
                    idx = _bcast_lane(base16, lane) + col
                    mrow = plsc.load_gather(
                        m_b[b], [jnp.broadcast_to(eoff + lane, (L,)), col]
                    )
                    old = plsc.load_gather(acc, [idx])
                    plsc.store_scatter(acc, [idx], jnp.maximum(old, mrow))

    pltpu.sync_copy(acc.at[pl.ds(0, nh * L)], out_hbm.at[h, g])


def _sc_scatter_max(m_p, dst, e, nh):
    eg = e // (NW // 2)
    mesh = plsc.VectorSubcoreMesh(
        core_axis_name="c", subcore_axis_name="s", num_cores=NC, num_subcores=NS
    )
    return pl.kernel(
        functools.partial(_scatter_body, eg, nh),
        out_type=jax.ShapeDtypeStruct((2, NW // 2, nh * L), jnp.float32),
        mesh=mesh,
        compiler_params=pltpu.CompilerParams(
            use_tc_tiling_on_sc=False, needs_layout_passes=False
        ),
        scratch_types=[
            pltpu.VMEM(((nh + 1) * L,), jnp.float32),
            pltpu.VMEM((1000,), jnp.int32),
            pltpu.VMEM((1000,), jnp.int32),
            pltpu.VMEM((1000, L), jnp.float32),
            pltpu.VMEM((1000, L), jnp.float32),
            pltpu.SemaphoreType.DMA,
            pltpu.SemaphoreType.DMA,
        ],
    )(m_p, dst)


def _update_body(x_ref, p_ref, wx_ref, wr_ref, b1_ref, w2_ref, b2_ref, o_ref):
    r = jnp.max(p_ref[0], axis=0)
    r = jnp.where(jnp.isfinite(r), r, 0.0)
    u = jnp.maximum(
        jnp.dot(x_ref[0], wx_ref[...], preferred_element_type=jnp.float32)
        + jnp.dot(r, wr_ref[...], preferred_element_type=jnp.float32)
        + b1_ref[...],
        0.0,
    )
    o_ref[0, ...] = (
        jnp.dot(u, w2_ref[...], preferred_element_type=jnp.float32) + b2_ref[...]
    )


def _update_mlp(x_p3, part_p, wx_bd, wr_bd, b1_t, w2_bd, b2_t):
    _, nhp, dxp = x_p3.shape
    ngrp = part_p.shape[1]
    hp = wr_bd.shape[1]
    dop = w2_bd.shape[1]
    return pl.pallas_call(
        _update_body,
        grid=(2,),
        in_specs=[
            pl.BlockSpec((1, nhp, dxp), lambda i: (i, 0, 0)),
            pl.BlockSpec((1, ngrp, nhp, hp), lambda i: (i, 0, 0, 0)),
            pl.BlockSpec((dxp, hp), lambda i: (0, 0)),
            pl.BlockSpec((hp, hp), lambda i: (0, 0)),
            pl.BlockSpec((1, hp), lambda i: (0, 0)),
            pl.BlockSpec((hp, dop), lambda i: (0, 0)),
            pl.BlockSpec((1, dop), lambda i: (0, 0)),
        ],
        out_specs=pl.BlockSpec((1, nhp, dop), lambda i: (i, 0, 0)),
        out_shape=jax.ShapeDtypeStruct((2, nhp, dop), jnp.float32),
    )(x_p3, part_p, wx_bd, wr_bd, b1_t, w2_bd, b2_t)


def kernel(x, edge_index, edge_attr, W_msg1, b_msg1, W_msg2, b_msg2,
           W_udt1, b_udt1, W_udt2, b_udt2):
    n, dx = x.shape
    e = edge_index.shape[1]
    de = edge_attr.shape[1]
    nh = n // 2

    src = edge_index[0]
    dst = edge_index[1]

    eye8 = jnp.eye(8, dtype=jnp.float32)

    def bd(w):
        return jnp.kron(eye8, w)

    def bt(b):
        return jnp.tile(b, 8)[None, :]

    x_p = x.reshape(n // 8, 8 * dx)
    ea_t = edge_attr.T

    p_p = _node_proj(x_p, bd(W_msg1[de:]), bt(b_msg1))
    qt = _edge_proj(ea_t, W_msg1[:de])
    z_pb = _sc_gather(p_p.reshape(n, L), src, qt, e)
    m_p = _msg_mlp(z_pb, bd(W_msg2), bt(b_msg2))
    partials = _sc_scatter_max(m_p, dst, e, nh)
    part_p = partials.reshape(2, NW // 2, nh * L // 128, 128)
    x_p3 = x.reshape(2, nh // 8, 8 * dx)
    out_p = _update_mlp(x_p3, part_p, bd(W_udt1[:dx]), bd(W_udt1[dx:]),
                        bt(b_udt1), bd(W_udt2), bt(b_udt2))
    return out_p.reshape(n, W_udt2.shape[1])


# speedup vs baseline: 1.1323x; 1.1323x over previous
import functools

import jax
import jax.numpy as jnp
from jax import lax
from jax.experimental import pallas as pl
from jax.experimental.pallas import tpu as pltpu
from jax.experimental.pallas import tpu_sc as plsc

NC = 2
NS = 16
NW = NC * NS
L = 16


def _proj_body(x_ref, w_ref, b_ref, o_ref):
    o_ref[...] = (
        jnp.dot(x_ref[...], w_ref[...], preferred_element_type=jnp.float32)
        + b_ref[...]
    )


def _node_proj(x_p, w_bd, b_t):
    np8, dxp = x_p.shape
    hp = w_bd.shape[1]
    return pl.pallas_call(
        _proj_body,
        grid=(1,),
        in_specs=[
            pl.BlockSpec((np8, dxp), lambda i: (0, 0)),
            pl.BlockSpec((dxp, hp), lambda i: (0, 0)),
            pl.BlockSpec((1, hp), lambda i: (0, 0)),
        ],
        out_specs=pl.BlockSpec((np8, hp), lambda i: (0, 0)),
        out_shape=jax.ShapeDtypeStruct((np8, hp), jnp.float32),
    )(x_p, w_bd, b_t)


def _banded(ebase):
    k = ebase // 1000
    return (k // 8) * 1000, (k % 8) * L


def _gather_body(ew, p_hbm, src_hbm, g_hbm, idx0, idx1, rows0, rows1,
                 sem_i0, sem_i1, sem_g, sem_w0, sem_w1):
    wid = lax.axis_index("s") * NC + lax.axis_index("c")
    nch = ew // 1000
    idx_b = (idx0, idx1)
    rows_b = (rows0, rows1)
    sem_i = (sem_i0, sem_i1)
    sem_w = (sem_w0, sem_w1)

    def fire_idx(ci, b):
        pltpu.async_copy(
            src_hbm.at[pl.ds(wid * ew + ci * 1000, 1000)], idx_b[b], sem_i[b]
        )

    fire_idx(0, 0)

    @pl.loop(0, nch // 2)
    def _outer(o):
        for b in range(2):
            ci = 2 * o + b

            @pl.when(ci + 1 < nch)
            def _():
                fire_idx(ci + 1, 1 - b)

            pltpu.make_async_copy(
                src_hbm.at[pl.ds(0, 1000)], idx_b[b], sem_i[b]
            ).wait()

            @pl.when(o > 0)
            def _():
                pltpu.make_async_copy(
                    rows_b[b], g_hbm.at[pl.ds(0, 1000), pl.ds(0, L)], sem_w[b]
                ).wait()

            copies = [
                pltpu.async_copy(
                    p_hbm.at[idx_b[b].at[pl.ds(j * 128, min(128, 1000 - j * 128))]],
                    rows_b[b].at[pl.ds(j * 128, min(128, 1000 - j * 128))],
                    sem_g,
                )
                for j in range(8)
            ]
            for cp in copies:
                cp.wait()
            row0, col0 = _banded(wid * ew + ci * 1000)
            pltpu.async_copy(
                rows_b[b],
                g_hbm.at[pl.ds(row0, 1000), pl.ds(col0, L)],
                sem_w[b],
            )

    for b in range(2):
        pltpu.make_async_copy(
            rows_b[b], g_hbm.at[pl.ds(0, 1000), pl.ds(0, L)], sem_w[b]
        ).wait()


def _sc_gather(p, src, e):
    ew = e // NW
    mesh = plsc.VectorSubcoreMesh(
        core_axis_name="c", subcore_axis_name="s", num_cores=NC, num_subcores=NS
    )
    return pl.kernel(
        functools.partial(_gather_body, ew),
        out_type=jax.ShapeDtypeStruct((e // 8, 8 * L), jnp.float32),
        mesh=mesh,
        compiler_params=pltpu.CompilerParams(use_tc_tiling_on_sc=False),
        scratch_types=[
            pltpu.VMEM((1000,), jnp.int32),
            pltpu.VMEM((1000,), jnp.int32),
            pltpu.VMEM((1000, L), jnp.float32),
            pltpu.VMEM((1000, L), jnp.float32),
            pltpu.SemaphoreType.DMA,
            pltpu.SemaphoreType.DMA,
            pltpu.SemaphoreType.DMA,
            pltpu.SemaphoreType.DMA,
            pltpu.SemaphoreType.DMA,
        ],
    )(p, src)


def _msg_body(eat_ref, g_ref, w1_ref, w2_ref, b2_ref, o_ref):
    q = lax.dot_general(
        eat_ref[...], w1_ref[...],
        (((0,), (0,)), ((), ())),
        preferred_element_type=jnp.float32,
    )
    nsub = q.shape[0] // 8000
    subs = [
        jnp.concatenate(
            [q[s * 8000 + j * 1000:s * 8000 + (j + 1) * 1000, :]
             for j in range(8)],
            axis=1,
        )
        for s in range(nsub)
    ]
    qp = jnp.concatenate(subs, axis=0) if nsub > 1 else subs[0]
    z = jnp.maximum(qp + g_ref[...], 0.0)
    o_ref[...] = (
        jnp.dot(z, w2_ref[...], preferred_element_type=jnp.float32) + b2_ref[...]
    )


def _msg_mlp(ea_t, g_pb, w1, w2_bd, b2_t):
    de, e = ea_t.shape
    ep8, dep = g_pb.shape
    blk = 2000
    return pl.pallas_call(
        _msg_body,
        grid=(ep8 // blk,),
        in_specs=[
            pl.BlockSpec((de, 8 * blk), lambda i: (0, i)),
            pl.BlockSpec((blk, dep), lambda i: (i, 0)),
            pl.BlockSpec((de, de), lambda i: (0, 0)),
            pl.BlockSpec((dep, dep), lambda i: (0, 0)),
            pl.BlockSpec((1, dep), lambda i: (0, 0)),
        ],
        out_specs=pl.BlockSpec((blk, dep), lambda i: (i, 0)),
        out_shape=jax.ShapeDtypeStruct((ep8, dep), jnp.float32),
        compiler_params=pltpu.CompilerParams(
            fuse_transposed_lhs_in_matmul=True
        ),
    )(ea_t, g_pb, w1, w2_bd, b2_t)


_BCAST_DNUMS = lax.GatherDimensionNumbers(
    offset_dims=(), collapsed_slice_dims=(0,), start_index_map=(0,)
)


def _bcast_lane(v, b):
    return lax.gather(
        v,
        jnp.full((L, 1), b, jnp.int32),
        _BCAST_DNUMS,
        slice_sizes=(1,),
        mode=lax.GatherScatterMode.PROMISE_IN_BOUNDS,
    )


def _scatter_body(eg, nh, m_hbm, dst_hbm, out_hbm, acc,
                  d0, d1, m0, m1, sem0, sem1):
    wid = lax.axis_index("s") * NC + lax.axis_index("c")
    g = wid // 2
    h = wid % 2
    col = lax.iota(jnp.int32, L)
    neg_inf = jnp.full((L,), -jnp.inf, dtype=jnp.float32)
    d_b = (d0, d1)
    m_b = (m0, m1)
    sem = (sem0, sem1)

    @pl.loop(0, nh + 1)
    def _init(i):
        acc[pl.ds(i * L, L)] = neg_inf

    nch = eg // 1000
    nvec = 1000 // L + 1

    def fire(ci, b):
        ebase = g * eg + ci * 1000
        row0, col0 = _banded(ebase)
        pltpu.async_copy(dst_hbm.at[pl.ds(ebase, 1000)], d_b[b], sem[b])
        pltpu.async_copy(
            m_hbm.at[pl.ds(row0, 1000), pl.ds(col0, L)], m_b[b], sem[b]
        )

    fire(0, 0)

    @pl.loop(0, nch // 2)
    def _outer(o):
        for b in range(2):
            ci = 2 * o + b

            @pl.when(ci + 1 < nch)
            def _():
                fire(ci + 1, 1 - b)

            pltpu.make_async_copy(
                dst_hbm.at[pl.ds(0, 1000)], d_b[b], sem[b]
            ).wait()
            pltpu.make_async_copy(
                m_hbm.at[pl.ds(0, 1000), pl.ds(0, L)], m_b[b], sem[b]
            ).wait()

            @pl.loop(0, nvec)
            def _vec(k):
                eoff = jnp.minimum(k * L, 1000 - L)
                dstv = d_b[b][pl.ds(eoff, L)]
                off = dstv - h * nh
                owned = (off >= 0) & (off < nh)
                base16 = jnp.where(owned, off, nh) * L
                for lane in range(L):
                    idx = _bcast_lane(base16, lane) + col
                    mrow = plsc.load_gather(
                        m_b[b], [jnp.broadcast_to(eoff + lane, (L,)), col]
                    )
                    old = plsc.load_gather(acc, [idx])
                    plsc.store_scatter(acc, [idx], jnp.maximum(old, mrow))

    pltpu.sync_copy(acc.at[pl.ds(0, nh * L)], out_hbm.at[h, g])


def _sc_scatter_max(m_p, dst, e, nh):
    eg = e // (NW // 2)
    mesh = plsc.VectorSubcoreMesh(
        core_axis_name="c", subcore_axis_name="s", num_cores=NC, num_subcores=NS
    )
    return pl.kernel(
        functools.partial(_scatter_body, eg, nh),
        out_type=jax.ShapeDtypeStruct((2, NW // 2, nh * L), jnp.float32),
        mesh=mesh,
        compiler_params=pltpu.CompilerParams(
            use_tc_tiling_on_sc=False, needs_layout_passes=False
        ),
        scratch_types=[
            pltpu.VMEM(((nh + 1) * L,), jnp.float32),
            pltpu.VMEM((1000,), jnp.int32),
            pltpu.VMEM((1000,), jnp.int32),
            pltpu.VMEM((1000, L), jnp.float32),
            pltpu.VMEM((1000, L), jnp.float32),
            pltpu.SemaphoreType.DMA,
            pltpu.SemaphoreType.DMA,
        ],
    )(m_p, dst)


def _update_body(x_ref, p_ref, wx_ref, wr_ref, b1_ref, w2_ref, b2_ref, o_ref):
    r = jnp.max(p_ref[0], axis=0)
    r = jnp.where(jnp.isfinite(r), r, 0.0)
    u = jnp.maximum(
        jnp.dot(x_ref[0], wx_ref[...], preferred_element_type=jnp.float32)
        + jnp.dot(r, wr_ref[...], preferred_element_type=jnp.float32)
        + b1_ref[...],
        0.0,
    )
    o_ref[0, ...] = (
        jnp.dot(u, w2_ref[...], preferred_element_type=jnp.float32) + b2_ref[...]
    )


def _update_mlp(x_p3, part_p, wx_bd, wr_bd, b1_t, w2_bd, b2_t):
    _, nhp, dxp = x_p3.shape
    ngrp = part_p.shape[1]
    hp = wr_bd.shape[1]
    dop = w2_bd.shape[1]
    return pl.pallas_call(
        _update_body,
        grid=(2,),
        in_specs=[
            pl.BlockSpec((1, nhp, dxp), lambda i: (i, 0, 0)),
            pl.BlockSpec((1, ngrp, nhp, hp), lambda i: (i, 0, 0, 0)),
            pl.BlockSpec((dxp, hp), lambda i: (0, 0)),
            pl.BlockSpec((hp, hp), lambda i: (0, 0)),
            pl.BlockSpec((1, hp), lambda i: (0, 0)),
            pl.BlockSpec((hp, dop), lambda i: (0, 0)),
            pl.BlockSpec((1, dop), lambda i: (0, 0)),
        ],
        out_specs=pl.BlockSpec((1, nhp, dop), lambda i: (i, 0, 0)),
        out_shape=jax.ShapeDtypeStruct((2, nhp, dop), jnp.float32),
    )(x_p3, part_p, wx_bd, wr_bd, b1_t, w2_bd, b2_t)


def kernel(x, edge_index, edge_attr, W_msg1, b_msg1, W_msg2, b_msg2,
           W_udt1, b_udt1, W_udt2, b_udt2):
    n, dx = x.shape
    e = edge_index.shape[1]
    de = edge_attr.shape[1]
    nh = n // 2

    src = edge_index[0]
    dst = edge_index[1]

    eye8 = jnp.eye(8, dtype=jnp.float32)

    def bd(w):
        return jnp.kron(eye8, w)

    def bt(b):
        return jnp.tile(b, 8)[None, :]

    x_p = x.reshape(n // 8, 8 * dx)
    ea_t = edge_attr.T

    p_p = _node_proj(x_p, bd(W_msg1[de:]), bt(b_msg1))
    g_pb = _sc_gather(p_p.reshape(n, L), src, e)
    m_p = _msg_mlp(ea_t, g_pb,
                   W_msg1[:de], bd(W_msg2), bt(b_msg2))
    partials = _sc_scatter_max(m_p, dst, e, nh)
    part_p = partials.reshape(2, NW // 2, nh * L // 128, 128)
    x_p3 = x.reshape(2, nh // 8, 8 * dx)
    out_p = _update_mlp(x_p3, part_p, bd(W_udt1[:dx]), bd(W_udt1[dx:]),
                        bt(b_udt1), bd(W_udt2), bt(b_udt2))
    return out_p.reshape(n, W_udt2.shape[1])
